# Initial kernel scaffold; baseline (speedup 1.0000x reference)
import jax
import jax.numpy as jnp
from jax.experimental import pallas as pl


def kernel(x, Wq, Wk, Wv, Wo, g1, b1, g2, b2, w_gate, ew1, eb1, ew2, eb2):
    raise NotImplementedError("write your pallas kernel here")


# speedup vs baseline: 1.2684x; 1.2684x over previous
import functools

import jax
import jax.numpy as jnp
from jax.experimental import pallas as pl
from jax.experimental.pallas import tpu as pltpu

B, N, D = 1, 2048, 1024
H = 16
HD = D // H
E = 8
K = 2
FF = 512
EPS = 1e-5


def _qkv_body(x_ref, w_ref, o_ref):
    o_ref[...] = jnp.dot(x_ref[...], w_ref[...], preferred_element_type=jnp.float32)


def _qkv(x, wqkv):
    CB = 512
    return pl.pallas_call(
        _qkv_body,
        grid=(3 * D // CB,),
        in_specs=[
            pl.BlockSpec((N, D), lambda c: (0, 0)),
            pl.BlockSpec((D, CB), lambda c: (0, c)),
        ],
        out_specs=pl.BlockSpec((N, CB), lambda c: (0, c)),
        out_shape=jax.ShapeDtypeStruct((N, 3 * D), jnp.float32),
    )(x, wqkv)


def _attn_body(q_ref, k_ref, v_ref, o_ref):
    for sub in range(2):
        q = q_ref[:, sub * HD:(sub + 1) * HD]
        k = k_ref[:, sub * HD:(sub + 1) * HD]
        v = v_ref[:, sub * HD:(sub + 1) * HD]
        s = jax.lax.dot_general(
            q, k, (((1,), (1,)), ((), ())),
            preferred_element_type=jnp.float32,
        ) * (1.0 / (HD ** 0.5))
        m = jnp.max(s, axis=1, keepdims=True)
        p = jnp.exp(s - m)
        p = p / jnp.sum(p, axis=1, keepdims=True)
        o_ref[:, sub * HD:(sub + 1) * HD] = jnp.dot(
            p, v, preferred_element_type=jnp.float32)


def _attn(qkv):
    RB = 512
    HP = H // 2
    return pl.pallas_call(
        _attn_body,
        grid=(HP, N // RB),
        in_specs=[
            pl.BlockSpec((RB, 2 * HD), lambda h, r: (r, h)),
            pl.BlockSpec((N, 2 * HD), lambda h, r: (0, HP + h)),
            pl.BlockSpec((N, 2 * HD), lambda h, r: (0, 2 * HP + h)),
        ],
        out_specs=pl.BlockSpec((RB, 2 * HD), lambda h, r: (r, h)),
        out_shape=jax.ShapeDtypeStruct((N, D), jnp.float32),
    )(qkv, qkv, qkv)


def _proj_norm_body(o_ref, wo_ref, x_ref, g_ref, b_ref, h_ref):
    t = jnp.dot(o_ref[...], wo_ref[...], preferred_element_type=jnp.float32) + x_ref[...]
    m = jnp.mean(t, axis=0, keepdims=True)
    v = jnp.mean((t - m) ** 2, axis=0, keepdims=True)
    h_ref[...] = (t - m) * jax.lax.rsqrt(v + EPS) * g_ref[...] + b_ref[...]


def _proj_norm(o, wo, x, g, b):
    CB = 128
    return pl.pallas_call(
        _proj_norm_body,
        grid=(D // CB,),
        in_specs=[
            pl.BlockSpec((N, D), lambda c: (0, 0)),
            pl.BlockSpec((D, CB), lambda c: (0, c)),
            pl.BlockSpec((N, CB), lambda c: (0, c)),
            pl.BlockSpec((1, CB), lambda c: (0, c)),
            pl.BlockSpec((1, CB), lambda c: (0, c)),
        ],
        out_specs=pl.BlockSpec((N, CB), lambda c: (0, c)),
        out_shape=jax.ShapeDtypeStruct((N, D), jnp.float32),
    )(o, wo, x, g, b)


def _gate_body(h_ref, wg_ref, gates_ref):
    logits = jnp.dot(h_ref[...], wg_ref[...], preferred_element_type=jnp.float32)
    logits = logits[:, :E]
    ii = jax.lax.broadcasted_iota(jnp.int32, (N, E), 1)
    m1 = jnp.max(logits, axis=1, keepdims=True)
    i1 = jnp.min(jnp.where(logits == m1, ii, E), axis=1, keepdims=True)
    mask1 = ii == i1
    l2 = jnp.where(mask1, -jnp.inf, logits)
    m2 = jnp.max(l2, axis=1, keepdims=True)
    i2 = jnp.min(jnp.where(l2 == m2, ii, E), axis=1, keepdims=True)
    mask2 = ii == i2
    e2 = jnp.exp(m2 - m1)
    g1 = 1.0 / (1.0 + e2)
    g2 = e2 * g1
    gates = jnp.where(mask1, g1, jnp.where(mask2, g2, 0.0))
    gates_ref[...] = jnp.pad(gates, ((0, 0), (0, 128 - E)))


def _gating(h, wg_pad):
    return pl.pallas_call(
        _gate_body,
        grid=(1,),
        in_specs=[
            pl.BlockSpec((N, D), lambda i: (0, 0)),
            pl.BlockSpec((D, 128), lambda i: (0, 0)),
        ],
        out_specs=pl.BlockSpec((N, 128), lambda i: (0, 0)),
        out_shape=jax.ShapeDtypeStruct((N, 128), jnp.float32),
    )(h, wg_pad)


def _moe_body(h_ref, w1_ref, b1_ref, w2_ref, b2_ref, gt_ref, y_ref):
    e = pl.program_id(0)
    hid = jnp.maximum(
        jnp.dot(h_ref[...], w1_ref[0], preferred_element_type=jnp.float32) + b1_ref[0], 0.0
    )
    out = jnp.dot(hid, w2_ref[0], preferred_element_type=jnp.float32) + b2_ref[0]
    g = gt_ref[0, 0, :].reshape(N, 1)
    contrib = g * out

    @pl.when(e == 0)
    def _():
        y_ref[...] = contrib

    @pl.when(e > 0)
    def _():
        y_ref[...] += contrib


def _moe_dense(h, ew1, eb1, ew2, eb2, gates_t):
    return pl.pallas_call(
        _moe_body,
        grid=(E,),
        in_specs=[
            pl.BlockSpec((N, D), lambda e: (0, 0)),
            pl.BlockSpec((1, D, FF), lambda e: (e, 0, 0)),
            pl.BlockSpec((1, 1, FF), lambda e: (e, 0, 0)),
            pl.BlockSpec((1, FF, D), lambda e: (e, 0, 0)),
            pl.BlockSpec((1, 1, D), lambda e: (e, 0, 0)),
            pl.BlockSpec((1, 1, N), lambda e: (e, 0, 0)),
        ],
        out_specs=pl.BlockSpec((N, D), lambda e: (0, 0)),
        out_shape=jax.ShapeDtypeStruct((N, D), jnp.float32),
    )(h, ew1, eb1, ew2, eb2, gates_t)


def _final_body(y_ref, h_ref, g_ref, b_ref, o_ref):
    t = y_ref[...] + h_ref[...]
    m = jnp.mean(t, axis=0, keepdims=True)
    v = jnp.mean((t - m) ** 2, axis=0, keepdims=True)
    o_ref[...] = (t - m) * jax.lax.rsqrt(v + EPS) * g_ref[...] + b_ref[...]


def _final_norm(y, h, g, b):
    CB = 128
    return pl.pallas_call(
        _final_body,
        grid=(D // CB,),
        in_specs=[
            pl.BlockSpec((N, CB), lambda c: (0, c)),
            pl.BlockSpec((N, CB), lambda c: (0, c)),
            pl.BlockSpec((1, CB), lambda c: (0, c)),
            pl.BlockSpec((1, CB), lambda c: (0, c)),
        ],
        out_specs=pl.BlockSpec((N, CB), lambda c: (0, c)),
        out_shape=jax.ShapeDtypeStruct((N, D), jnp.float32),
    )(y, h, g, b)


def kernel(x, Wq, Wk, Wv, Wo, g1, b1, g2, b2, w_gate, ew1, eb1, ew2, eb2):
    x2 = x.reshape(N, D)
    wqkv = jnp.concatenate([Wq, Wk, Wv], axis=1)
    qkv = _qkv(x2, wqkv)
    o = _attn(qkv)
    h = _proj_norm(o, Wo, x2, g1.reshape(1, D), b1.reshape(1, D))
    wg_pad = jnp.pad(w_gate, ((0, 0), (0, 128 - E)))
    gates = _gating(h, wg_pad)[:, :E]
    gates_t = gates.T.reshape(E, 1, N)
    y = _moe_dense(
        h, ew1, eb1.reshape(E, 1, FF), ew2, eb2.reshape(E, 1, D), gates_t
    )
    h2 = _final_norm(y, h, g2.reshape(1, D), b2.reshape(1, D))
    return h2.reshape(B, N, D)
